# Initial kernel scaffold; baseline (speedup 1.0000x reference)
import jax
import jax.numpy as jnp
from jax.experimental import pallas as pl


def kernel(toks, pos, W_E, W_pos):
    raise NotImplementedError("write your pallas kernel here")


# speedup vs baseline: 1.3486x; 1.3486x over previous
import functools

import jax
import jax.numpy as jnp
from jax import lax
from jax.experimental import pallas as pl
from jax.experimental.pallas import tpu as pltpu
from jax.experimental.pallas import tpu_sc as plsc

D_MODEL = 768
N_TOKENS = 8192
NC, NS, L = 2, 16, 16
NW = NC * NS
PER_W = N_TOKENS // NW
CHUNK = 64
NCHUNK = PER_W // CHUNK
VECS = D_MODEL // L


def _emb_kernel(toks_hbm, pos_hbm, we_hbm, wpos_hbm, out_hbm,
                tok_idx, pos_idx, tokbuf, posbuf, sem1, sem2):
    wid = lax.axis_index("s") * NC + lax.axis_index("c")
    base = wid * PER_W

    pltpu.sync_copy(toks_hbm.at[wid], tok_idx)
    pltpu.sync_copy(pos_hbm.at[wid], pos_idx)

    for g in range(NCHUNK):
        c1 = pltpu.async_copy(we_hbm.at[tok_idx.at[g]], tokbuf, sem1)
        c2 = pltpu.async_copy(wpos_hbm.at[pos_idx.at[g]], posbuf, sem2)
        c1.wait()
        c2.wait()

        def row_body(r, carry):
            for j in range(VECS):
                sl = pl.ds(j * L, L)
                tokbuf[r, sl] = tokbuf[r, sl] + posbuf[r, sl]
            return carry

        lax.fori_loop(0, CHUNK, row_body, 0)
        pltpu.sync_copy(tokbuf, out_hbm.at[pl.ds(base + g * CHUNK, CHUNK)])


@jax.jit
def kernel(toks, pos, W_E, W_pos):
    B, S = toks.shape
    toks32 = toks.reshape(NW, NCHUNK, CHUNK).astype(jnp.int32)
    pos32 = pos.reshape(NW, NCHUNK, CHUNK).astype(jnp.int32)

    run = functools.partial(
        pl.kernel,
        out_type=jax.ShapeDtypeStruct((N_TOKENS, D_MODEL), jnp.float32),
        mesh=plsc.VectorSubcoreMesh(core_axis_name="c", subcore_axis_name="s"),
        scratch_types=[
            pltpu.VMEM((NCHUNK, CHUNK), jnp.int32),
            pltpu.VMEM((NCHUNK, CHUNK), jnp.int32),
            pltpu.VMEM((CHUNK, D_MODEL), jnp.float32),
            pltpu.VMEM((CHUNK, D_MODEL), jnp.float32),
            pltpu.SemaphoreType.DMA,
            pltpu.SemaphoreType.DMA,
        ],
    )(_emb_kernel)
    flat = run(toks32, pos32, W_E, W_pos)
    return flat.reshape(B, S, D_MODEL)


# speedup vs baseline: 1.6255x; 1.2054x over previous
import functools

import jax
import jax.numpy as jnp
from jax import lax
from jax.experimental import pallas as pl
from jax.experimental.pallas import tpu as pltpu
from jax.experimental.pallas import tpu_sc as plsc

D_MODEL = 768
N_TOKENS = 8192
NC, NS, L = 2, 16, 16
NW = NC * NS
PER_W = N_TOKENS // NW
CHUNK = 32
NCHUNK = PER_W // CHUNK
VECS = D_MODEL // L


def _emb_kernel(toks_hbm, pos_hbm, we_hbm, wpos_hbm, out_hbm,
                tok_idx, pos_idx,
                tokbuf0, posbuf0, tokbuf1, posbuf1,
                gsem0, gsem1, wsem0, wsem1):
    wid = lax.axis_index("s") * NC + lax.axis_index("c")
    base = wid * PER_W

    tokbufs = (tokbuf0, tokbuf1)
    posbufs = (posbuf0, posbuf1)
    gsems = (gsem0, gsem1)
    wsems = (wsem0, wsem1)

    pltpu.sync_copy(toks_hbm.at[wid], tok_idx)
    pltpu.sync_copy(pos_hbm.at[wid], pos_idx)

    def fire(g, slot):
        return (
            pltpu.async_copy(we_hbm.at[tok_idx.at[g]], tokbufs[slot], gsems[slot]),
            pltpu.async_copy(wpos_hbm.at[pos_idx.at[g]], posbufs[slot], gsems[slot]),
        )

    gh = [None, None]
    wh = [None, None]
    gh[0] = fire(0, 0)
    for g in range(NCHUNK):
        cur = g % 2
        nxt = 1 - cur
        if g + 1 < NCHUNK:
            if wh[nxt] is not None:
                wh[nxt].wait()
            gh[nxt] = fire(g + 1, nxt)

        gh[cur][0].wait()
        gh[cur][1].wait()

        tb, pb = tokbufs[cur], posbufs[cur]

        def row_body(r, carry):
            for j in range(VECS):
                sl = pl.ds(j * L, L)
                tb[r, sl] = tb[r, sl] + pb[r, sl]
            return carry

        lax.fori_loop(0, CHUNK, row_body, 0)

        wh[cur] = pltpu.async_copy(
            tb, out_hbm.at[pl.ds(base + g * CHUNK, CHUNK)], wsems[cur])

    wh[0].wait()
    wh[1].wait()


@jax.jit
def kernel(toks, pos, W_E, W_pos):
    B, S = toks.shape
    toks32 = toks.reshape(NW, NCHUNK, CHUNK).astype(jnp.int32)
    pos32 = pos.reshape(NW, NCHUNK, CHUNK).astype(jnp.int32)

    run = functools.partial(
        pl.kernel,
        out_type=jax.ShapeDtypeStruct((N_TOKENS, D_MODEL), jnp.float32),
        mesh=plsc.VectorSubcoreMesh(core_axis_name="c", subcore_axis_name="s"),
        scratch_types=[
            pltpu.VMEM((NCHUNK, CHUNK), jnp.int32),
            pltpu.VMEM((NCHUNK, CHUNK), jnp.int32),
            pltpu.VMEM((CHUNK, D_MODEL), jnp.float32),
            pltpu.VMEM((CHUNK, D_MODEL), jnp.float32),
            pltpu.VMEM((CHUNK, D_MODEL), jnp.float32),
            pltpu.VMEM((CHUNK, D_MODEL), jnp.float32),
            pltpu.SemaphoreType.DMA,
            pltpu.SemaphoreType.DMA,
            pltpu.SemaphoreType.DMA,
            pltpu.SemaphoreType.DMA,
        ],
    )(_emb_kernel)
    flat = run(toks32, pos32, W_E, W_pos)
    return flat.reshape(B, S, D_MODEL)
